# baseline (speedup 1.0000x reference)
import jax
import jax.numpy as jnp
import numpy as np
from jax import lax
from jax.experimental import pallas as pl
from jax.experimental.pallas import tpu as pltpu
from jax.experimental.pallas import tpu_sc as plsc

_K = 64
_IMIN = np.int32(-2147483648)
_NSLOT = 4


def _keys(v):
    bits = lax.bitcast_convert_type(v, jnp.int32)
    return bits ^ (lax.shift_right_arithmetic(bits, 31) & np.int32(0x7FFFFFFF))


def _sc_body(x_hbm, o_hbm, xv, ov, h0, h1, h2, h3, sem):
    hists = (h0, h1, h2, h3)
    nc = 2
    wid = lax.axis_index("s") * nc + lax.axis_index("c")
    n = 32768
    n_chunks = n // 16
    lane = lax.iota(jnp.int32, 16)

    def do_row(r, _):
        row = wid * 4 + r
        pltpu.sync_copy(x_hbm.at[row], xv)

        def level_step(lv, carry):
            prefix_u, kneed = carry
            shift = 24 - 8 * lv
            mask_hi = jnp.where(lv == 0, np.int32(0),
                                lax.shift_left(np.int32(-1),
                                               jnp.minimum(shift + 8, 31)))

            def zero_step(i, _):
                z = jnp.zeros((16,), jnp.float32)
                for h in hists:
                    h[pl.ds(i * 16, 16)] = z
                return 0
            lax.fori_loop(0, 256, zero_step, 0)

            def hist_step(i, _):
                for j in range(_NSLOT):
                    ci = i * _NSLOT + j
                    s = _keys(xv[pl.ds(ci * 16, 16)])
                    u = s ^ _IMIN
                    match = (u & mask_hi) == prefix_u
                    bucket = lax.shift_right_logical(u, shift) & np.int32(0xFF)
                    idxv = bucket * 16 + lane
                    h = plsc.load_gather(hists[j], [idxv])
                    plsc.store_scatter(hists[j], [idxv],
                                       h + jnp.where(match, 1.0, 0.0))
                return 0
            lax.fori_loop(0, n_chunks // _NSLOT, hist_step, 0)

            def scan_cond(c):
                b, cum, bfound, cum_before = c
                return (bfound < 0) & (b >= 0)

            def scan_body(c):
                b, cum, bfound, cum_before = c
                t = (h0[pl.ds(b * 16, 16)] + h1[pl.ds(b * 16, 16)]
                     + h2[pl.ds(b * 16, 16)] + h3[pl.ds(b * 16, 16)])
                cnt = jnp.sum(t).astype(jnp.int32)
                cum_new = cum + cnt
                hit = cum_new >= kneed
                return (b - 1, cum_new,
                        jnp.where(hit, b, bfound),
                        jnp.where(hit, cum, cum_before))

            _, _, bsel, cum_before = lax.while_loop(
                scan_cond, scan_body,
                (np.int32(255), np.int32(0), np.int32(-1), np.int32(0)))

            prefix_u = prefix_u | lax.shift_left(bsel, shift)
            kneed = kneed - cum_before
            return prefix_u, kneed

        t_u, kneed = lax.fori_loop(0, 4, level_step,
                                   (np.int32(0), np.int32(_K)))
        t_s = t_u ^ _IMIN

        def out_step(i, c_eq):
            cc = c_eq
            for j in range(_NSLOT):
                ci = i * _NSLOT + j
                v = xv[pl.ds(ci * 16, 16)]
                s = _keys(v)
                gt = s > t_s
                eqm = s == t_s
                cnt = plsc.all_reduce_population_count(eqm)[0]
                rem = kneed - cc

                def full_prefix(_):
                    pfx = plsc.cumsum(jnp.where(eqm, 1, 0))
                    return gt | (eqm & (pfx <= rem))

                def cheap(_):
                    return gt | (eqm & (rem >= cnt))

                boundary = (rem > 0) & (rem < cnt)
                keep = lax.cond(boundary, full_prefix, cheap, 0)
                ov[pl.ds(ci * 16, 16)] = jnp.where(
                    keep, jnp.maximum(v, 0.0), jnp.float32(0.0))
                cc = cc + cnt
            return cc
        lax.fori_loop(0, n_chunks // _NSLOT, out_step, np.int32(0))

        pltpu.sync_copy(ov, o_hbm.at[row])
        return 0

    lax.fori_loop(0, 4, do_row, 0)


def kernel(x):
    n_rows, n = x.shape
    mesh = plsc.VectorSubcoreMesh(core_axis_name="c", subcore_axis_name="s",
                                  num_cores=2, num_subcores=16)
    return pl.kernel(
        _sc_body,
        out_type=jax.ShapeDtypeStruct((n_rows, n), jnp.float32),
        mesh=mesh,
        compiler_params=pltpu.CompilerParams(needs_layout_passes=False),
        scratch_types=[
            pltpu.VMEM((n,), jnp.float32),
            pltpu.VMEM((n,), jnp.float32),
            pltpu.VMEM((4096,), jnp.float32),
            pltpu.VMEM((4096,), jnp.float32),
            pltpu.VMEM((4096,), jnp.float32),
            pltpu.VMEM((4096,), jnp.float32),
            pltpu.SemaphoreType.DMA,
        ],
    )(x)


# speedup vs baseline: 1.2545x; 1.2545x over previous
import jax
import jax.numpy as jnp
import numpy as np
from jax import lax
from jax.experimental import pallas as pl
from jax.experimental.pallas import tpu as pltpu
from jax.experimental.pallas import tpu_sc as plsc

_K = 64
_IMIN = np.int32(-2147483648)
_NSLOT = 4


def _keys(v):
    bits = lax.bitcast_convert_type(v, jnp.int32)
    return bits ^ (lax.shift_right_arithmetic(bits, 31) & np.int32(0x7FFFFFFF))


def _sc_body(x_hbm, o_hbm, xv, ov, h0, h1, h2, h3, sem):
    hists = (h0, h1, h2, h3)
    nc = 2
    wid = lax.axis_index("s") * nc + lax.axis_index("c")
    n = 32768
    n_chunks = n // 16
    lane = lax.iota(jnp.int32, 16)

    def do_row(r, _):
        row = wid * 4 + r
        pltpu.sync_copy(x_hbm.at[row], xv)

        def level_step(lv, carry):
            prefix_u, kneed = carry
            shift = 24 - 8 * lv
            mask_hi = jnp.where(lv == 0, np.int32(0),
                                lax.shift_left(np.int32(-1),
                                               jnp.minimum(shift + 8, 31)))

            def zero_step(i, _):
                z = jnp.zeros((16,), jnp.float32)
                for h in hists:
                    h[pl.ds(i * 16, 16)] = z
                return 0
            lax.fori_loop(0, 256, zero_step, 0)

            def hist_step(i, _):
                for j in range(_NSLOT):
                    ci = i * _NSLOT + j
                    s = _keys(xv[pl.ds(ci * 16, 16)])
                    u = s ^ _IMIN
                    match = (u & mask_hi) == prefix_u
                    bucket = lax.shift_right_logical(u, shift) & np.int32(0xFF)
                    idxv = bucket * 16 + lane
                    h = plsc.load_gather(hists[j], [idxv])
                    plsc.store_scatter(hists[j], [idxv],
                                       h + jnp.where(match, 1.0, 0.0))
                return 0
            lax.fori_loop(0, n_chunks // _NSLOT, hist_step, 0)

            def scan_cond(c):
                b, cum, bfound, cum_before = c
                return (bfound < 0) & (b >= 0)

            def scan_body(c):
                b, cum, bfound, cum_before = c
                t = (h0[pl.ds(b * 16, 16)] + h1[pl.ds(b * 16, 16)]
                     + h2[pl.ds(b * 16, 16)] + h3[pl.ds(b * 16, 16)])
                cnt = jnp.sum(t).astype(jnp.int32)
                cum_new = cum + cnt
                hit = cum_new >= kneed
                return (b - 1, cum_new,
                        jnp.where(hit, b, bfound),
                        jnp.where(hit, cum, cum_before))

            _, _, bsel, cum_before = lax.while_loop(
                scan_cond, scan_body,
                (np.int32(255), np.int32(0), np.int32(-1), np.int32(0)))

            prefix_u = prefix_u | lax.shift_left(bsel, shift)
            kneed = kneed - cum_before
            return prefix_u, kneed

        t_u, kneed = lax.fori_loop(0, 4, level_step,
                                   (np.int32(0), np.int32(_K)))
        t_s = t_u ^ _IMIN

        def out_step(i, c_eq):
            cc = c_eq
            for j in range(_NSLOT):
                ci = i * _NSLOT + j
                v = xv[pl.ds(ci * 16, 16)]
                s = _keys(v)
                gt = s > t_s
                eqm = s == t_s
                cnt = plsc.all_reduce_population_count(eqm)[0]
                rem = kneed - cc
                pfx = plsc.cumsum(jnp.where(eqm, 1, 0))
                keep = gt | (eqm & (pfx <= rem))
                ov[pl.ds(ci * 16, 16)] = jnp.where(
                    keep, jnp.maximum(v, 0.0), jnp.float32(0.0))
                cc = cc + cnt
            return cc
        lax.fori_loop(0, n_chunks // _NSLOT, out_step, np.int32(0))

        pltpu.sync_copy(ov, o_hbm.at[row])
        return 0

    lax.fori_loop(0, 4, do_row, 0)


def kernel(x):
    n_rows, n = x.shape
    mesh = plsc.VectorSubcoreMesh(core_axis_name="c", subcore_axis_name="s",
                                  num_cores=2, num_subcores=16)
    return pl.kernel(
        _sc_body,
        out_type=jax.ShapeDtypeStruct((n_rows, n), jnp.float32),
        mesh=mesh,
        compiler_params=pltpu.CompilerParams(needs_layout_passes=False),
        scratch_types=[
            pltpu.VMEM((n,), jnp.float32),
            pltpu.VMEM((n,), jnp.float32),
            pltpu.VMEM((4096,), jnp.float32),
            pltpu.VMEM((4096,), jnp.float32),
            pltpu.VMEM((4096,), jnp.float32),
            pltpu.VMEM((4096,), jnp.float32),
            pltpu.SemaphoreType.DMA,
        ],
    )(x)


# speedup vs baseline: 2.2302x; 1.7778x over previous
import jax
import jax.numpy as jnp
import numpy as np
from jax import lax
from jax.experimental import pallas as pl
from jax.experimental.pallas import tpu as pltpu
from jax.experimental.pallas import tpu_sc as plsc

_K = 64
_IMIN = np.int32(-2147483648)
_NSLOT = 4
_ROWS_PER_BLOCK = 8
_LANES = 128
_SC_ROWS = 32


def _tc_body(x_ref, o_ref):
    x = x_ref[...]
    r_dim, g_dim, l_dim = x.shape
    bits = lax.bitcast_convert_type(x, jnp.int32)
    s = bits ^ (lax.shift_right_arithmetic(bits, 31) & jnp.int32(0x7FFFFFFF))
    imin = jnp.int32(-2147483648)

    def count_ge(cand_s):
        m = (s >= cand_s).astype(jnp.int32)
        return jnp.sum(m, axis=(1, 2)).reshape(r_dim, 1, 1)

    def bit_step(i, pu):
        cand_u = pu | (jnp.int32(1) << (31 - i))
        cnt = count_ge(cand_u ^ imin)
        return jnp.where(cnt >= _K, cand_u, pu)

    pu = lax.fori_loop(0, 32, bit_step, jnp.zeros((r_dim, 1, 1), jnp.int32))
    t_s = pu ^ imin

    gt = s > t_s
    eq = s == t_s
    count_gt = jnp.sum(gt.astype(jnp.int32), axis=(1, 2)).reshape(r_dim, 1, 1)
    need = _K - count_gt

    gidx = lax.broadcasted_iota(jnp.int32, x.shape, 1)

    def g_step(i, carry):
        lo, hi = carry
        mid = (lo + hi) >> 1
        cnt = jnp.sum(jnp.where(eq & (gidx <= mid), 1, 0),
                      axis=(1, 2)).reshape(r_dim, 1, 1)
        pred = cnt >= need
        return jnp.where(pred, lo, mid + 1), jnp.where(pred, mid, hi)

    g_bits = (g_dim - 1).bit_length()
    h, _ = lax.fori_loop(0, g_bits, g_step,
                         (jnp.zeros((r_dim, 1, 1), jnp.int32),
                          jnp.full((r_dim, 1, 1), g_dim - 1, jnp.int32)))
    cnt_before = jnp.sum(jnp.where(eq & (gidx < h), 1, 0),
                         axis=(1, 2)).reshape(r_dim, 1, 1)
    need2 = (need - cnt_before).astype(jnp.float32)

    tri_l = (lax.broadcasted_iota(jnp.int32, (l_dim, l_dim), 0)
             <= lax.broadcasted_iota(jnp.int32, (l_dim, l_dim), 1)
             ).astype(jnp.float32)
    eq2 = eq.astype(jnp.float32).reshape(r_dim * g_dim, l_dim)
    lane_pfx = jnp.dot(eq2, tri_l,
                       preferred_element_type=jnp.float32
                       ).reshape(r_dim, g_dim, l_dim)

    keep_eq = eq & ((gidx < h) | ((gidx == h) & (lane_pfx <= need2)))
    keep = gt | keep_eq
    o_ref[...] = jnp.where(keep, jnp.maximum(x, 0.0), jnp.float32(0.0))


def _tc_kernel(x):
    n_rows, n = x.shape
    g = n // _LANES
    xr = x.reshape(n_rows, g, _LANES)
    out = pl.pallas_call(
        _tc_body,
        grid=(n_rows // _ROWS_PER_BLOCK,),
        in_specs=[pl.BlockSpec((_ROWS_PER_BLOCK, g, _LANES),
                               lambda i: (i, 0, 0))],
        out_specs=pl.BlockSpec((_ROWS_PER_BLOCK, g, _LANES),
                               lambda i: (i, 0, 0)),
        out_shape=jax.ShapeDtypeStruct((n_rows, g, _LANES), jnp.float32),
    )(xr)
    return out.reshape(n_rows, n)


def _keys(v):
    bits = lax.bitcast_convert_type(v, jnp.int32)
    return bits ^ (lax.shift_right_arithmetic(bits, 31) & np.int32(0x7FFFFFFF))


def _sc_body(x_hbm, o_hbm, xv, ov, h0, h1, h2, h3, sem):
    hists = (h0, h1, h2, h3)
    nc = 2
    wid = lax.axis_index("s") * nc + lax.axis_index("c")
    n = 32768
    n_chunks = n // 16
    lane = lax.iota(jnp.int32, 16)

    row = wid
    pltpu.sync_copy(x_hbm.at[row], xv)

    def level_step(lv, carry):
        prefix_u, kneed = carry
        shift = 24 - 8 * lv
        mask_hi = jnp.where(lv == 0, np.int32(0),
                            lax.shift_left(np.int32(-1),
                                           jnp.minimum(shift + 8, 31)))

        def zero_step(i, _):
            z = jnp.zeros((16,), jnp.float32)
            for h in hists:
                h[pl.ds(i * 16, 16)] = z
            return 0
        lax.fori_loop(0, 256, zero_step, 0)

        def hist_step(i, _):
            for j in range(_NSLOT):
                ci = i * _NSLOT + j
                s = _keys(xv[pl.ds(ci * 16, 16)])
                u = s ^ _IMIN
                match = (u & mask_hi) == prefix_u
                bucket = lax.shift_right_logical(u, shift) & np.int32(0xFF)
                idxv = bucket * 16 + lane
                h = plsc.load_gather(hists[j], [idxv])
                plsc.store_scatter(hists[j], [idxv],
                                   h + jnp.where(match, 1.0, 0.0))
            return 0
        lax.fori_loop(0, n_chunks // _NSLOT, hist_step, 0)

        def scan_cond(c):
            b, cum, bfound, cum_before = c
            return (bfound < 0) & (b >= 0)

        def scan_body(c):
            b, cum, bfound, cum_before = c
            t = (h0[pl.ds(b * 16, 16)] + h1[pl.ds(b * 16, 16)]
                 + h2[pl.ds(b * 16, 16)] + h3[pl.ds(b * 16, 16)])
            cnt = jnp.sum(t).astype(jnp.int32)
            cum_new = cum + cnt
            hit = cum_new >= kneed
            return (b - 1, cum_new,
                    jnp.where(hit, b, bfound),
                    jnp.where(hit, cum, cum_before))

        _, _, bsel, cum_before = lax.while_loop(
            scan_cond, scan_body,
            (np.int32(255), np.int32(0), np.int32(-1), np.int32(0)))

        prefix_u = prefix_u | lax.shift_left(bsel, shift)
        kneed = kneed - cum_before
        return prefix_u, kneed

    t_u, kneed = lax.fori_loop(0, 4, level_step,
                               (np.int32(0), np.int32(_K)))
    t_s = t_u ^ _IMIN

    def out_step(i, c_eq):
        cc = c_eq
        for j in range(_NSLOT):
            ci = i * _NSLOT + j
            v = xv[pl.ds(ci * 16, 16)]
            s = _keys(v)
            gt = s > t_s
            eqm = s == t_s
            cnt = plsc.all_reduce_population_count(eqm)[0]
            rem = kneed - cc
            pfx = plsc.cumsum(jnp.where(eqm, 1, 0))
            keep = gt | (eqm & (pfx <= rem))
            ov[pl.ds(ci * 16, 16)] = jnp.where(
                keep, jnp.maximum(v, 0.0), jnp.float32(0.0))
            cc = cc + cnt
        return cc
    lax.fori_loop(0, n_chunks // _NSLOT, out_step, np.int32(0))

    pltpu.sync_copy(ov, o_hbm.at[row])


def _sc_kernel(x):
    n_rows, n = x.shape
    mesh = plsc.VectorSubcoreMesh(core_axis_name="c", subcore_axis_name="s",
                                  num_cores=2, num_subcores=16)
    return pl.kernel(
        _sc_body,
        out_type=jax.ShapeDtypeStruct((n_rows, n), jnp.float32),
        mesh=mesh,
        compiler_params=pltpu.CompilerParams(needs_layout_passes=False),
        scratch_types=[
            pltpu.VMEM((n,), jnp.float32),
            pltpu.VMEM((n,), jnp.float32),
            pltpu.VMEM((4096,), jnp.float32),
            pltpu.VMEM((4096,), jnp.float32),
            pltpu.VMEM((4096,), jnp.float32),
            pltpu.VMEM((4096,), jnp.float32),
            pltpu.SemaphoreType.DMA,
        ],
    )(x)


def kernel(x):
    n_rows = x.shape[0]
    n_tc = n_rows - _SC_ROWS
    out_tc = _tc_kernel(x[:n_tc])
    out_sc = _sc_kernel(x[n_tc:])
    return jnp.concatenate([out_tc, out_sc], axis=0)


# speedup vs baseline: 3.7874x; 1.6982x over previous
import jax
import jax.numpy as jnp
from jax import lax
from jax.experimental import pallas as pl

_K = 64
_ROWS_PER_BLOCK = 8
_LANES = 128


def _topk_mask_body(x_ref, o_ref):
    x = x_ref[...]
    r_dim, g_dim, l_dim = x.shape
    bits = lax.bitcast_convert_type(x, jnp.int32)
    s = bits ^ (lax.shift_right_arithmetic(bits, 31) & jnp.int32(0x7FFFFFFF))
    imin = jnp.int32(-2147483648)

    u = s ^ imin
    s_hi = (lax.shift_right_logical(u, 16) ^ jnp.int32(0x8000)
            ).astype(jnp.int16)
    one16 = jnp.bfloat16(1)
    zero16 = jnp.bfloat16(0)

    def count16(mask):
        p = jnp.sum(jnp.where(mask, one16, zero16), axis=1,
                    dtype=jnp.bfloat16)
        return jnp.sum(p.astype(jnp.int32), axis=1).reshape(r_dim, 1, 1)

    def hi_step(i, ph):
        cand_u = ph | (jnp.int32(1) << (15 - i))
        cand16 = (cand_u ^ jnp.int32(0x8000)).astype(jnp.int16)
        cnt = count16(s_hi >= cand16)
        return jnp.where(cnt >= _K, cand_u, ph)

    t_hi = lax.fori_loop(0, 16, hi_step, jnp.zeros((r_dim, 1, 1), jnp.int32))
    t_hi16 = (t_hi ^ jnp.int32(0x8000)).astype(jnp.int16)

    cnt_gt_hi = count16(s_hi > t_hi16)
    kprime = _K - cnt_gt_hi

    m_hi = s_hi == t_hi16
    s_lo = ((u & jnp.int32(0xFFFF)) ^ jnp.int32(0x8000)).astype(jnp.int16)

    def lo_step(i, pli):
        cand_u = pli | (jnp.int32(1) << (15 - i))
        cand16 = (cand_u ^ jnp.int32(0x8000)).astype(jnp.int16)
        cnt = count16(m_hi & (s_lo >= cand16))
        return jnp.where(cnt >= kprime, cand_u, pli)

    t_lo = lax.fori_loop(0, 16, lo_step, jnp.zeros((r_dim, 1, 1), jnp.int32))

    t_s = (lax.shift_left(t_hi, 16) | t_lo) ^ imin

    gt = s > t_s
    eq = s == t_s
    count_gt = jnp.sum(gt.astype(jnp.int32), axis=(1, 2)).reshape(r_dim, 1, 1)
    need = _K - count_gt

    gidx = lax.broadcasted_iota(jnp.int32, x.shape, 1)

    def g_step(i, carry):
        lo, hi = carry
        mid = (lo + hi) >> 1
        cnt = jnp.sum(jnp.where(eq & (gidx <= mid), 1, 0),
                      axis=(1, 2)).reshape(r_dim, 1, 1)
        pred = cnt >= need
        return jnp.where(pred, lo, mid + 1), jnp.where(pred, mid, hi)

    g_bits = (g_dim - 1).bit_length()
    h, _ = lax.fori_loop(0, g_bits, g_step,
                         (jnp.zeros((r_dim, 1, 1), jnp.int32),
                          jnp.full((r_dim, 1, 1), g_dim - 1, jnp.int32)))
    cnt_before = jnp.sum(jnp.where(eq & (gidx < h), 1, 0),
                         axis=(1, 2)).reshape(r_dim, 1, 1)
    need2 = (need - cnt_before).astype(jnp.float32)

    tri_l = (lax.broadcasted_iota(jnp.int32, (l_dim, l_dim), 0)
             <= lax.broadcasted_iota(jnp.int32, (l_dim, l_dim), 1)
             ).astype(jnp.float32)
    eq2 = eq.astype(jnp.float32).reshape(r_dim * g_dim, l_dim)
    lane_pfx = jnp.dot(eq2, tri_l,
                       preferred_element_type=jnp.float32
                       ).reshape(r_dim, g_dim, l_dim)

    keep_eq = eq & ((gidx < h) | ((gidx == h) & (lane_pfx <= need2)))
    keep = gt | keep_eq
    o_ref[...] = jnp.where(keep, jnp.maximum(x, 0.0), jnp.float32(0.0))


def kernel(x):
    n_rows, n = x.shape
    g = n // _LANES
    xr = x.reshape(n_rows, g, _LANES)
    out = pl.pallas_call(
        _topk_mask_body,
        grid=(n_rows // _ROWS_PER_BLOCK,),
        in_specs=[pl.BlockSpec((_ROWS_PER_BLOCK, g, _LANES), lambda i: (i, 0, 0))],
        out_specs=pl.BlockSpec((_ROWS_PER_BLOCK, g, _LANES), lambda i: (i, 0, 0)),
        out_shape=jax.ShapeDtypeStruct((n_rows, g, _LANES), jnp.float32),
    )(xr)
    return out.reshape(n_rows, n)


# speedup vs baseline: 4.7594x; 1.2567x over previous
import jax
import jax.numpy as jnp
from jax import lax
from jax.experimental import pallas as pl

_K = 64
_ROWS_PER_BLOCK = 16
_LANES = 128


def _topk_mask_body(x_ref, o_ref):
    x = x_ref[...]
    r_dim, g_dim, l_dim = x.shape
    bits = lax.bitcast_convert_type(x, jnp.int32)
    s = bits ^ (lax.shift_right_arithmetic(bits, 31) & jnp.int32(0x7FFFFFFF))
    imin = jnp.int32(-2147483648)

    u = s ^ imin
    s_hi = (lax.shift_right_logical(u, 16) ^ jnp.int32(0x8000)
            ).astype(jnp.int16)
    one16 = jnp.bfloat16(1)
    zero16 = jnp.bfloat16(0)

    def count16(mask):
        p = jnp.sum(jnp.where(mask, one16, zero16), axis=1,
                    dtype=jnp.bfloat16)
        return jnp.sum(p.astype(jnp.int32), axis=1).reshape(r_dim, 1, 1)

    def hi_step(i, ph):
        cand_u = ph | (jnp.int32(1) << (15 - i))
        cand16 = (cand_u ^ jnp.int32(0x8000)).astype(jnp.int16)
        cnt = count16(s_hi >= cand16)
        return jnp.where(cnt >= _K, cand_u, ph)

    t_hi = lax.fori_loop(0, 16, hi_step, jnp.zeros((r_dim, 1, 1), jnp.int32))
    t_hi16 = (t_hi ^ jnp.int32(0x8000)).astype(jnp.int16)

    cnt_gt_hi = count16(s_hi > t_hi16)
    kprime = _K - cnt_gt_hi

    m_hi = s_hi == t_hi16
    s_lo = ((u & jnp.int32(0xFFFF)) ^ jnp.int32(0x8000)).astype(jnp.int16)

    def lo_step(i, pli):
        cand_u = pli | (jnp.int32(1) << (15 - i))
        cand16 = (cand_u ^ jnp.int32(0x8000)).astype(jnp.int16)
        cnt = count16(m_hi & (s_lo >= cand16))
        return jnp.where(cnt >= kprime, cand_u, pli)

    t_lo = lax.fori_loop(0, 16, lo_step, jnp.zeros((r_dim, 1, 1), jnp.int32))

    t_s = (lax.shift_left(t_hi, 16) | t_lo) ^ imin

    gt = s > t_s
    eq = s == t_s
    count_gt = jnp.sum(gt.astype(jnp.int32), axis=(1, 2)).reshape(r_dim, 1, 1)
    need = _K - count_gt

    gidx = lax.broadcasted_iota(jnp.int32, x.shape, 1)
    eqf = eq.astype(jnp.float32)
    gsum = jnp.sum(eqf, axis=2)
    giota = lax.broadcasted_iota(jnp.int32, (r_dim, g_dim), 1)
    needf2 = need.astype(jnp.float32).reshape(r_dim, 1)

    def g_step(i, carry):
        lo, hi = carry
        mid = (lo + hi) >> 1
        cnt = jnp.sum(jnp.where(giota <= mid, gsum, 0.0),
                      axis=1).reshape(r_dim, 1)
        pred = cnt >= needf2
        return jnp.where(pred, lo, mid + 1), jnp.where(pred, mid, hi)

    g_bits = (g_dim - 1).bit_length()
    h2, _ = lax.fori_loop(0, g_bits, g_step,
                          (jnp.zeros((r_dim, 1), jnp.int32),
                           jnp.full((r_dim, 1), g_dim - 1, jnp.int32)))
    cnt_before = jnp.sum(jnp.where(giota < h2, gsum, 0.0),
                         axis=1).reshape(r_dim, 1, 1)
    h = h2.reshape(r_dim, 1, 1)
    need2 = need.astype(jnp.float32) - cnt_before

    tri_l = (lax.broadcasted_iota(jnp.int32, (l_dim, l_dim), 0)
             <= lax.broadcasted_iota(jnp.int32, (l_dim, l_dim), 1)
             ).astype(jnp.float32)
    eq2 = eq.astype(jnp.float32).reshape(r_dim * g_dim, l_dim)
    lane_pfx = jnp.dot(eq2, tri_l,
                       preferred_element_type=jnp.float32
                       ).reshape(r_dim, g_dim, l_dim)

    keep_eq = eq & ((gidx < h) | ((gidx == h) & (lane_pfx <= need2)))
    keep = gt | keep_eq
    o_ref[...] = jnp.where(keep, jnp.maximum(x, 0.0), jnp.float32(0.0))


def kernel(x):
    n_rows, n = x.shape
    g = n // _LANES
    xr = x.reshape(n_rows, g, _LANES)
    out = pl.pallas_call(
        _topk_mask_body,
        grid=(n_rows // _ROWS_PER_BLOCK,),
        in_specs=[pl.BlockSpec((_ROWS_PER_BLOCK, g, _LANES), lambda i: (i, 0, 0))],
        out_specs=pl.BlockSpec((_ROWS_PER_BLOCK, g, _LANES), lambda i: (i, 0, 0)),
        out_shape=jax.ShapeDtypeStruct((n_rows, g, _LANES), jnp.float32),
    )(xr)
    return out.reshape(n_rows, n)


# speedup vs baseline: 5.1572x; 1.0836x over previous
import jax
import jax.numpy as jnp
from jax import lax
from jax.experimental import pallas as pl

_K = 64
_ROWS_PER_BLOCK = 32
_LANES = 128


def _topk_mask_body(x_ref, o_ref):
    x = x_ref[...]
    r_dim, g_dim, l_dim = x.shape
    bits = lax.bitcast_convert_type(x, jnp.int32)
    s = bits ^ (lax.shift_right_arithmetic(bits, 31) & jnp.int32(0x7FFFFFFF))
    imin = jnp.int32(-2147483648)

    u = s ^ imin
    s_hi = (lax.shift_right_logical(u, 16) ^ jnp.int32(0x8000)
            ).astype(jnp.int16)
    one16 = jnp.bfloat16(1)
    zero16 = jnp.bfloat16(0)

    def count16(mask):
        p = jnp.sum(jnp.where(mask, one16, zero16), axis=1,
                    dtype=jnp.bfloat16)
        return jnp.sum(p.astype(jnp.int32), axis=1).reshape(r_dim, 1, 1)

    def hi_step(i, ph):
        cand_u = ph | (jnp.int32(1) << (15 - i))
        cand16 = (cand_u ^ jnp.int32(0x8000)).astype(jnp.int16)
        cnt = count16(s_hi >= cand16)
        return jnp.where(cnt >= _K, cand_u, ph)

    t_hi = lax.fori_loop(0, 16, hi_step, jnp.zeros((r_dim, 1, 1), jnp.int32))
    t_hi16 = (t_hi ^ jnp.int32(0x8000)).astype(jnp.int16)

    cnt_gt_hi = count16(s_hi > t_hi16)
    kprime = _K - cnt_gt_hi

    m_hi = s_hi == t_hi16
    s_lo = ((u & jnp.int32(0xFFFF)) ^ jnp.int32(0x8000)).astype(jnp.int16)

    def lo_step(i, pli):
        cand_u = pli | (jnp.int32(1) << (15 - i))
        cand16 = (cand_u ^ jnp.int32(0x8000)).astype(jnp.int16)
        cnt = count16(m_hi & (s_lo >= cand16))
        return jnp.where(cnt >= kprime, cand_u, pli)

    t_lo = lax.fori_loop(0, 16, lo_step, jnp.zeros((r_dim, 1, 1), jnp.int32))

    t_s = (lax.shift_left(t_hi, 16) | t_lo) ^ imin

    gt = s > t_s
    eq = s == t_s
    count_gt = jnp.sum(gt.astype(jnp.int32), axis=(1, 2)).reshape(r_dim, 1, 1)
    need = _K - count_gt

    gidx = lax.broadcasted_iota(jnp.int32, x.shape, 1)
    eqf = eq.astype(jnp.float32)
    gsum = jnp.sum(eqf, axis=2)
    giota = lax.broadcasted_iota(jnp.int32, (r_dim, g_dim), 1)
    needf2 = need.astype(jnp.float32).reshape(r_dim, 1)

    def g_step(i, carry):
        lo, hi = carry
        mid = (lo + hi) >> 1
        cnt = jnp.sum(jnp.where(giota <= mid, gsum, 0.0),
                      axis=1).reshape(r_dim, 1)
        pred = cnt >= needf2
        return jnp.where(pred, lo, mid + 1), jnp.where(pred, mid, hi)

    g_bits = (g_dim - 1).bit_length()
    h2, _ = lax.fori_loop(0, g_bits, g_step,
                          (jnp.zeros((r_dim, 1), jnp.int32),
                           jnp.full((r_dim, 1), g_dim - 1, jnp.int32)))
    cnt_before = jnp.sum(jnp.where(giota < h2, gsum, 0.0),
                         axis=1).reshape(r_dim, 1, 1)
    h = h2.reshape(r_dim, 1, 1)
    need2 = need.astype(jnp.float32) - cnt_before

    tri_l = (lax.broadcasted_iota(jnp.int32, (l_dim, l_dim), 0)
             <= lax.broadcasted_iota(jnp.int32, (l_dim, l_dim), 1)
             ).astype(jnp.float32)
    eq2 = eq.astype(jnp.float32).reshape(r_dim * g_dim, l_dim)
    lane_pfx = jnp.dot(eq2, tri_l,
                       preferred_element_type=jnp.float32
                       ).reshape(r_dim, g_dim, l_dim)

    keep_eq = eq & ((gidx < h) | ((gidx == h) & (lane_pfx <= need2)))
    keep = gt | keep_eq
    o_ref[...] = jnp.where(keep, jnp.maximum(x, 0.0), jnp.float32(0.0))


def kernel(x):
    n_rows, n = x.shape
    g = n // _LANES
    xr = x.reshape(n_rows, g, _LANES)
    out = pl.pallas_call(
        _topk_mask_body,
        grid=(n_rows // _ROWS_PER_BLOCK,),
        in_specs=[pl.BlockSpec((_ROWS_PER_BLOCK, g, _LANES), lambda i: (i, 0, 0))],
        out_specs=pl.BlockSpec((_ROWS_PER_BLOCK, g, _LANES), lambda i: (i, 0, 0)),
        out_shape=jax.ShapeDtypeStruct((n_rows, g, _LANES), jnp.float32),
    )(xr)
    return out.reshape(n_rows, n)


# speedup vs baseline: 5.3765x; 1.0425x over previous
import jax
import jax.numpy as jnp
from jax import lax
from jax.experimental import pallas as pl

_K = 64
_ROWS_PER_BLOCK = 32
_LANES = 128


def _topk_mask_body(x_ref, o_ref):
    x = x_ref[...]
    r_dim, g_dim, l_dim = x.shape
    bits = lax.bitcast_convert_type(x, jnp.int32)
    s = bits ^ (lax.shift_right_arithmetic(bits, 31) & jnp.int32(0x7FFFFFFF))
    imin = jnp.int32(-2147483648)

    u = s ^ imin
    s_hi = (lax.shift_right_logical(u, 16) ^ jnp.int32(0x8000)
            ).astype(jnp.int16)
    one16 = jnp.bfloat16(1)
    zero16 = jnp.bfloat16(0)

    def count16(mask):
        p = jnp.sum(jnp.where(mask, one16, zero16), axis=1,
                    dtype=jnp.bfloat16)
        return jnp.sum(p.astype(jnp.int32), axis=1).reshape(r_dim, 1, 1)

    def hi_step(i, ph):
        cand_u = ph | (jnp.int32(1) << (15 - i))
        cand16 = (cand_u ^ jnp.int32(0x8000)).astype(jnp.int16)
        cnt = count16(s_hi >= cand16)
        return jnp.where(cnt >= _K, cand_u, ph)

    t_hi = lax.fori_loop(0, 16, hi_step, jnp.zeros((r_dim, 1, 1), jnp.int32))
    t_hi16 = (t_hi ^ jnp.int32(0x8000)).astype(jnp.int16)

    cnt_gt_hi = count16(s_hi > t_hi16)
    kprime = _K - cnt_gt_hi

    m_hi = s_hi == t_hi16
    s_lo = ((u & jnp.int32(0xFFFF)) ^ jnp.int32(0x8000)).astype(jnp.int16)
    s_lo_m = jnp.where(m_hi, s_lo, jnp.int16(-32768))

    def lo_step(i, pli):
        cand_u = pli | (jnp.int32(1) << (15 - i))
        cand16 = (cand_u ^ jnp.int32(0x8000)).astype(jnp.int16)
        cnt = count16(s_lo_m >= cand16)
        return jnp.where(cnt >= kprime, cand_u, pli)

    t_lo = lax.fori_loop(0, 16, lo_step, jnp.zeros((r_dim, 1, 1), jnp.int32))

    t_s = (lax.shift_left(t_hi, 16) | t_lo) ^ imin

    gt = s > t_s
    eq = s == t_s
    count_gt = jnp.sum(gt.astype(jnp.int32), axis=(1, 2)).reshape(r_dim, 1, 1)
    need = _K - count_gt

    gidx = lax.broadcasted_iota(jnp.int32, x.shape, 1)
    eqf = eq.astype(jnp.float32)
    gsum = jnp.sum(eqf, axis=2)
    giota = lax.broadcasted_iota(jnp.int32, (r_dim, g_dim), 1)
    needf2 = need.astype(jnp.float32).reshape(r_dim, 1)

    def g_step(i, carry):
        lo, hi = carry
        mid = (lo + hi) >> 1
        cnt = jnp.sum(jnp.where(giota <= mid, gsum, 0.0),
                      axis=1).reshape(r_dim, 1)
        pred = cnt >= needf2
        return jnp.where(pred, lo, mid + 1), jnp.where(pred, mid, hi)

    g_bits = (g_dim - 1).bit_length()
    h2, _ = lax.fori_loop(0, g_bits, g_step,
                          (jnp.zeros((r_dim, 1), jnp.int32),
                           jnp.full((r_dim, 1), g_dim - 1, jnp.int32)))
    cnt_before = jnp.sum(jnp.where(giota < h2, gsum, 0.0),
                         axis=1).reshape(r_dim, 1, 1)
    h = h2.reshape(r_dim, 1, 1)
    need2 = need.astype(jnp.float32) - cnt_before

    tri_l = (lax.broadcasted_iota(jnp.int32, (l_dim, l_dim), 0)
             <= lax.broadcasted_iota(jnp.int32, (l_dim, l_dim), 1)
             ).astype(jnp.float32)
    eq2 = eq.astype(jnp.float32).reshape(r_dim * g_dim, l_dim)
    lane_pfx = jnp.dot(eq2, tri_l,
                       preferred_element_type=jnp.float32
                       ).reshape(r_dim, g_dim, l_dim)

    keep_eq = eq & ((gidx < h) | ((gidx == h) & (lane_pfx <= need2)))
    keep = gt | keep_eq
    o_ref[...] = jnp.where(keep, jnp.maximum(x, 0.0), jnp.float32(0.0))


def kernel(x):
    n_rows, n = x.shape
    g = n // _LANES
    xr = x.reshape(n_rows, g, _LANES)
    out = pl.pallas_call(
        _topk_mask_body,
        grid=(n_rows // _ROWS_PER_BLOCK,),
        in_specs=[pl.BlockSpec((_ROWS_PER_BLOCK, g, _LANES), lambda i: (i, 0, 0))],
        out_specs=pl.BlockSpec((_ROWS_PER_BLOCK, g, _LANES), lambda i: (i, 0, 0)),
        out_shape=jax.ShapeDtypeStruct((n_rows, g, _LANES), jnp.float32),
    )(xr)
    return out.reshape(n_rows, n)
